# speedup vs baseline: 1.1739x; 1.1739x over previous
import functools

import jax
import jax.numpy as jnp
from jax import lax
from jax.experimental import pallas as pl
from jax.experimental.pallas import tpu as pltpu
from jax.experimental.pallas import tpu_sc as plsc

NUM_TOKENS = 100
HIDDEN = 1024
BATCH = 1024
B_TOTAL = BATCH * NUM_TOKENS

NC = 2
NS = 16
NW = NC * NS
B_PER_W = B_TOTAL // NW
CHUNK = 40
NCHUNK = B_PER_W // CHUNK
NBUF = 2

_mesh = plsc.VectorSubcoreMesh(
    core_axis_name="c", subcore_axis_name="s", num_cores=NC, num_subcores=NS
)


@functools.partial(
    pl.kernel,
    out_type=jax.ShapeDtypeStruct((B_TOTAL, HIDDEN), jnp.float32),
    mesh=_mesh,
    scratch_types=[
        pltpu.VMEM((B_PER_W,), jnp.int32),
        pltpu.VMEM((NBUF, CHUNK, HIDDEN), jnp.float32),
        pltpu.SemaphoreType.DMA,
    ],
)
def _gather_kernel(idx_hbm, table_hbm, out_hbm, idx_v, rows_v, sem):
    wid = lax.axis_index("s") * NC + lax.axis_index("c")
    base = wid * B_PER_W
    pltpu.sync_copy(idx_hbm.at[pl.ds(base, B_PER_W)], idx_v)

    def outer(g):
        cps = []
        for b in range(NBUF):
            start = (g + b) * CHUNK
            cp = pltpu.async_copy(
                table_hbm.at[idx_v.at[pl.ds(start, CHUNK)]], rows_v.at[b], sem
            )
            cps.append(cp)
        for b in range(NBUF):
            start = (g + b) * CHUNK
            cps[b].wait()
            pltpu.sync_copy(
                rows_v.at[b], out_hbm.at[pl.ds(base + start, CHUNK)]
            )

    pl.loop(0, NCHUNK, step=NBUF)(outer)


def kernel(prompts, prompt_table):
    idx = prompts.reshape(-1).astype(jnp.int32)
    out = _gather_kernel(idx, prompt_table)
    return out.reshape(BATCH, NUM_TOKENS, HIDDEN)
